# Initial kernel scaffold; baseline (speedup 1.0000x reference)
import jax
import jax.numpy as jnp
from jax.experimental import pallas as pl


def kernel(scores, boxes):
    raise NotImplementedError("write your pallas kernel here")


# speedup vs baseline: 19.5666x; 19.5666x over previous
import functools

import jax
import jax.numpy as jnp
from jax import lax
from jax.experimental import pallas as pl
from jax.experimental.pallas import tpu as pltpu

_B = 8
_N = 5000
_C = 80
_D = 100
_NEG = -1e30


def _reduce_body(s_ref, best_ref, cls_ref):
    s = s_ref[0]
    m = jnp.max(s, axis=0, keepdims=True)
    ci = lax.broadcasted_iota(jnp.int32, (_C, _N), 0)
    c = jnp.min(jnp.where(s == m, ci, _C), axis=0, keepdims=True)
    best_ref[0] = m
    cls_ref[0] = c


def _nms_body(best_ref, cls_ref, boxes_ref,
              idx_ref, sc_ref, x1_ref, y1_ref, x2_ref, y2_ref, cls_out_ref,
              cnt_ref):
    best = best_ref[...]
    clsv = cls_ref[...]
    x1 = boxes_ref[0]
    y1 = boxes_ref[1]
    x2 = boxes_ref[2]
    y2 = boxes_ref[3]
    areas = (x2 - x1) * (y2 - y1)

    active0 = jnp.where(best > 0.0, best, _NEG)
    col = lax.broadcasted_iota(jnp.int32, (_B, _N), 1)
    ocol = lax.broadcasted_iota(jnp.int32, (_B, 128), 1)

    zf = jnp.zeros((_B, 128), jnp.float32)
    init = (active0,
            jnp.zeros((_B, 1), jnp.int32),
            jnp.full((_B, 128), -1, jnp.int32),
            zf,
            zf, zf, zf, zf,
            jnp.full((_B, 128), -1, jnp.int32))

    def step(t, carry):
        active, cnt, oidx, osc, ox1, oy1, ox2, oy2, ocls = carry
        m = jnp.max(active, axis=1, keepdims=True)
        valid = m > -1e29
        jj = jnp.min(jnp.where(active == m, col, _N), axis=1,
                     keepdims=True)
        onehot = col == jj

        def gf(v):
            return jnp.sum(jnp.where(onehot, v, 0.0), axis=1, keepdims=True)

        sx1 = gf(x1)
        sy1 = gf(y1)
        sx2 = gf(x2)
        sy2 = gf(y2)
        ssc = gf(best)
        sar = gf(areas)
        scl = jnp.sum(jnp.where(onehot, clsv, 0), axis=1, keepdims=True)

        iw = jnp.minimum(sx2, x2) - jnp.maximum(sx1, x1)
        ih = jnp.minimum(sy2, y2) - jnp.maximum(sy1, y1)
        inter = jnp.maximum(iw, 0.0) * jnp.maximum(ih, 0.0)
        union = sar + areas - inter
        supp = (inter > 0.5 * union) | onehot
        active = jnp.where(supp & valid, _NEG, active)
        cnt = cnt + valid.astype(jnp.int32)

        slot = ocol == t
        oidx = jnp.where(slot, jnp.where(valid, jj, -1), oidx)
        osc = jnp.where(slot, jnp.where(valid, ssc, 0.0), osc)
        ox1 = jnp.where(slot, jnp.where(valid, sx1, 0.0), ox1)
        oy1 = jnp.where(slot, jnp.where(valid, sy1, 0.0), oy1)
        ox2 = jnp.where(slot, jnp.where(valid, sx2, 0.0), ox2)
        oy2 = jnp.where(slot, jnp.where(valid, sy2, 0.0), oy2)
        ocls = jnp.where(slot, jnp.where(valid, scl, -1), ocls)
        return (active, cnt, oidx, osc, ox1, oy1, ox2, oy2, ocls)

    (_, cnt, oidx, osc, ox1, oy1, ox2, oy2, ocls) = lax.fori_loop(
        0, _D, step, init)

    idx_ref[...] = oidx[:, :_D]
    sc_ref[...] = osc[:, :_D]
    x1_ref[...] = ox1[:, :_D]
    y1_ref[...] = oy1[:, :_D]
    x2_ref[...] = ox2[:, :_D]
    y2_ref[...] = oy2[:, :_D]
    cls_out_ref[...] = ocls[:, :_D]
    cnt_ref[...] = cnt


@jax.jit
def kernel(scores, boxes):
    scores_t = jnp.swapaxes(scores, 1, 2)
    best, cls = pl.pallas_call(
        _reduce_body,
        grid=(_B,),
        in_specs=[pl.BlockSpec((1, _C, _N), lambda b: (b, 0, 0))],
        out_specs=[pl.BlockSpec((1, 1, _N), lambda b: (b, 0, 0)),
                   pl.BlockSpec((1, 1, _N), lambda b: (b, 0, 0))],
        out_shape=[jax.ShapeDtypeStruct((_B, 1, _N), jnp.float32),
                   jax.ShapeDtypeStruct((_B, 1, _N), jnp.int32)],
    )(scores_t)
    best = best.reshape(_B, _N)
    cls = cls.reshape(_B, _N)
    boxes_t = jnp.transpose(boxes, (2, 0, 1))

    outs = pl.pallas_call(
        _nms_body,
        out_shape=[jax.ShapeDtypeStruct((_B, _D), jnp.int32),
                   jax.ShapeDtypeStruct((_B, _D), jnp.float32),
                   jax.ShapeDtypeStruct((_B, _D), jnp.float32),
                   jax.ShapeDtypeStruct((_B, _D), jnp.float32),
                   jax.ShapeDtypeStruct((_B, _D), jnp.float32),
                   jax.ShapeDtypeStruct((_B, _D), jnp.float32),
                   jax.ShapeDtypeStruct((_B, _D), jnp.int32),
                   jax.ShapeDtypeStruct((_B, 1), jnp.int32)],
    )(best, cls, boxes_t)
    oidx, osc, ox1, oy1, ox2, oy2, ocls, cnt = outs
    boxes_out = jnp.stack([ox1, oy1, ox2, oy2], axis=-1)
    return oidx, osc, boxes_out, ocls, cnt.reshape(_B)
